# Initial kernel scaffold; baseline (speedup 1.0000x reference)
import jax
import jax.numpy as jnp
from jax.experimental import pallas as pl


def kernel(uEmbeds, iEmbeds, edge_weight, edge_index):
    raise NotImplementedError("write your pallas kernel here")


# speedup vs baseline: 4.2728x; 4.2728x over previous
import functools

import jax
import jax.numpy as jnp
from jax import lax
from jax.experimental import pallas as pl
from jax.experimental.pallas import tpu as pltpu
from jax.experimental.pallas import tpu_sc as plsc

NCORES = 2
NSUB = 16
NW = NCORES * NSUB
LANES = 16
NUSER = 6000
NNODES = 10000
NPAD = 10240
D = 128
NEDGES = 320000
C = 128
K = 79
EW = K * C
EPAD = NW * EW
ROWS_T = NPAD // NSUB
DUMP = NNODES

_mesh = plsc.VectorSubcoreMesh(core_axis_name="c", subcore_axis_name="s")


@functools.partial(
    pl.kernel,
    out_type=jax.ShapeDtypeStruct((NCORES, NPAD, D), jnp.float32),
    mesh=_mesh,
    scratch_types=[
        pltpu.VMEM((K, C), jnp.int32),
        pltpu.VMEM((K, C), jnp.int32),
        pltpu.VMEM((EW,), jnp.float32),
        pltpu.VMEM((C, D), jnp.float32),
        pltpu.VMEM_SHARED((NPAD, D), jnp.float32),
        pltpu.SemaphoreType.DMA,
    ],
)
def _spmm_partials(x_hbm, src_hbm, dst_hbm, w_hbm, out_hbm,
                   src_v, dst_v, w_v, rows_v, acc_s, sem):
    c = lax.axis_index("c")
    s = lax.axis_index("s")
    w_id = c * NSUB + s

    pltpu.sync_copy(src_hbm.at[w_id], src_v)
    pltpu.sync_copy(dst_hbm.at[w_id], dst_v)
    pltpu.sync_copy(w_hbm.at[w_id], w_v)

    zero = jnp.zeros((LANES,), jnp.float32)

    @plsc.parallel_loop(0, C)
    def _zero_rows(e):
        row = rows_v.at[e]
        for d in range(D // LANES):
            row[pl.ds(d * LANES, LANES)] = zero

    for r in range(ROWS_T // C):
        pltpu.sync_copy(rows_v, acc_s.at[pl.ds(s * ROWS_T + r * C, C)])

    plsc.subcore_barrier()

    def chunk_body(j, carry):
        pltpu.async_copy(x_hbm.at[src_v.at[j]], rows_v, sem).wait()

        @plsc.parallel_loop(0, C // LANES)
        def _scale(g):
            wv = w_v[pl.ds(j * C + g * LANES, LANES)]
            for le in range(LANES):
                wspl = wv.at[jnp.full((LANES,), le, jnp.int32)].get(
                    mode="promise_in_bounds")
                row = rows_v.at[g * LANES + le]
                for d in range(D // LANES):
                    sl = pl.ds(d * LANES, LANES)
                    row[sl] = row[sl] * wspl

        pltpu.sync_copy(rows_v, acc_s.at[dst_v.at[j]], add=True)
        return carry

    lax.fori_loop(0, K, chunk_body, 0)

    plsc.subcore_barrier()

    for r in range(ROWS_T // C):
        base = s * ROWS_T + r * C
        pltpu.sync_copy(acc_s.at[pl.ds(base, C)], rows_v)
        pltpu.sync_copy(rows_v, out_hbm.at[c].at[pl.ds(base, C)])


_BLK = 512


def _sum_body(*refs):
    out = refs[-1]
    acc = refs[0][...]
    for r in refs[1:-1]:
        acc = acc + r[...]
    out[...] = acc


def _tc_sum(arrs):
    n = len(arrs)
    return pl.pallas_call(
        _sum_body,
        out_shape=jax.ShapeDtypeStruct((NPAD, D), jnp.float32),
        grid=(NPAD // _BLK,),
        in_specs=[pl.BlockSpec((_BLK, D), lambda i: (i, 0))] * n,
        out_specs=pl.BlockSpec((_BLK, D), lambda i: (i, 0)),
    )(*arrs)


def kernel(uEmbeds, iEmbeds, edge_weight, edge_index):
    x0 = jnp.concatenate([uEmbeds, iEmbeds], axis=0)
    x0p = jnp.pad(x0, ((0, NPAD - NNODES), (0, 0)))
    src = edge_index[1].astype(jnp.int32)
    dst = edge_index[0].astype(jnp.int32)
    w = edge_weight.astype(jnp.float32)
    pad = EPAD - NEDGES
    src = jnp.pad(src, (0, pad)).reshape(NW, K, C)
    dst = jnp.pad(dst, (0, pad), constant_values=DUMP).reshape(NW, K, C)
    w = jnp.pad(w, (0, pad)).reshape(NW, EW)

    p = _spmm_partials(x0p, src, dst, w)
    x1 = _tc_sum([p[0], p[1]])
    q = _spmm_partials(x1, src, dst, w)
    out = _tc_sum([x0p, x1, q[0], q[1]])
    return (out[:NUSER], out[NUSER:NNODES])
